# baseline (speedup 1.0000x reference)
import functools

import jax
import jax.numpy as jnp
from jax import lax
from jax.experimental import pallas as pl
from jax.experimental.pallas import tpu as pltpu
from jax.experimental.pallas import tpu_sc as plsc

_NC = 2
_NS = 16
_NW = _NC * _NS


def _fcg_body(rows_per_w, rows_last, ch, nchunk,
              ci_hbm, c1_hbm, c2_hbm, c3_hbm, o1_hbm, o2_hbm, o3_hbm,
              ci_v, c1_v, c2_v, c3_v, o1_v, o2_v, o3_v):
    wid = lax.axis_index("s") * _NC + lax.axis_index("c")
    base = wid * rows_per_w
    nrows = jnp.where(wid == _NW - 1, rows_last, rows_per_w)
    last_start = base + nrows - ch

    n_vregs = ch // 16

    def chunk_body(i, carry):
        start = jnp.minimum(base + i * ch, last_start)
        pltpu.sync_copy(ci_hbm.at[pl.ds(start, ch)], ci_v)
        pltpu.sync_copy(c1_hbm.at[pl.ds(start, ch)], c1_v)
        pltpu.sync_copy(c2_hbm.at[pl.ds(start, ch)], c2_v)
        pltpu.sync_copy(c3_hbm.at[pl.ds(start, ch)], c3_v)

        def vreg_body(g, c2_):
            s = pl.ds(g * 16, 16)
            civ = ci_v[s]
            o1_v[s] = (c1_v[s] << 1) + (civ & 1)
            o2_v[s] = (c2_v[s] << 1) + ((civ >> 1) & 1)
            o3_v[s] = (c3_v[s] << 1) + ((civ >> 2) & 1)
            return c2_

        lax.fori_loop(0, n_vregs, vreg_body, 0)
        pltpu.sync_copy(o1_v, o1_hbm.at[pl.ds(start, ch)])
        pltpu.sync_copy(o2_v, o2_hbm.at[pl.ds(start, ch)])
        pltpu.sync_copy(o3_v, o3_hbm.at[pl.ds(start, ch)])
        return carry

    lax.fori_loop(0, nchunk, chunk_body, 0)


def _copy_block(x_ref, o_ref):
    o_ref[...] = x_ref[...]


def _dma_copy_body(x_hbm, o_hbm, *sems):
    n = len(sems)
    rows = x_hbm.shape[0] // n
    cps = [
        pltpu.make_async_copy(x_hbm.at[pl.ds(k * rows, rows)],
                              o_hbm.at[pl.ds(k * rows, rows)], sems[k])
        for k in range(n)
    ]
    for cp in cps:
        cp.start()
    for cp in cps:
        cp.wait()


def kernel(parent_coords, child_indices, parent_features):
    m = parent_coords.shape[0]
    assert m % 16 == 0 and m // _NW >= 16
    rows_per_w = (m // _NW) // 16 * 16
    rows_last = m - (_NW - 1) * rows_per_w
    ch = min(8192, rows_per_w) // 16 * 16
    nchunk = -(-rows_last // ch)

    ci = child_indices.astype(jnp.int32)

    body = functools.partial(_fcg_body, rows_per_w, rows_last, ch, nchunk)
    o1, o2, o3 = pl.kernel(
        body,
        out_type=[jax.ShapeDtypeStruct((m,), jnp.int32) for _ in range(3)],
        mesh=plsc.VectorSubcoreMesh(core_axis_name="c", subcore_axis_name="s"),
        scratch_types=[pltpu.VMEM((ch,), jnp.int32) for _ in range(7)],
    )(ci, parent_coords[:, 1], parent_coords[:, 2], parent_coords[:, 3])
    c, mm = parent_features.shape[1], m
    nd = 8
    pfT = pl.pallas_call(
        _dma_copy_body,
        out_shape=jax.ShapeDtypeStruct((c, mm), parent_features.dtype),
        in_specs=[pl.BlockSpec(memory_space=pl.ANY)],
        out_specs=pl.BlockSpec(memory_space=pl.ANY),
        scratch_shapes=[pltpu.SemaphoreType.DMA] * nd,
    )(parent_features.T)
    pf = pfT.T
    o1, o2, o3, pf = lax.optimization_barrier((o1, o2, o3, pf))
    out = jnp.stack([parent_coords[:, 0], o1, o2, o3], axis=1)
    return out, pf


# speedup vs baseline: 23.8241x; 23.8241x over previous
import functools

import jax
import jax.numpy as jnp
from jax import lax
from jax.experimental import pallas as pl
from jax.experimental.pallas import tpu as pltpu
from jax.experimental.pallas import tpu_sc as plsc

_NC = 2
_NS = 16
_NW = _NC * _NS


def _fcg_body(rows_per_w, rows_last, ch, nchunk,
              ci_hbm, c1_hbm, c2_hbm, c3_hbm, o1_hbm, o2_hbm, o3_hbm,
              ci_v, c1_v, c2_v, c3_v, o1_v, o2_v, o3_v):
    wid = lax.axis_index("s") * _NC + lax.axis_index("c")
    base = wid * rows_per_w
    nrows = jnp.where(wid == _NW - 1, rows_last, rows_per_w)
    last_start = base + nrows - ch

    n_vregs = ch // 16

    def chunk_body(i, carry):
        start = jnp.minimum(base + i * ch, last_start)
        pltpu.sync_copy(ci_hbm.at[pl.ds(start, ch)], ci_v)
        pltpu.sync_copy(c1_hbm.at[pl.ds(start, ch)], c1_v)
        pltpu.sync_copy(c2_hbm.at[pl.ds(start, ch)], c2_v)
        pltpu.sync_copy(c3_hbm.at[pl.ds(start, ch)], c3_v)

        def vreg_body(g, c2_):
            s = pl.ds(g * 16, 16)
            civ = ci_v[s]
            o1_v[s] = (c1_v[s] << 1) + (civ & 1)
            o2_v[s] = (c2_v[s] << 1) + ((civ >> 1) & 1)
            o3_v[s] = (c3_v[s] << 1) + ((civ >> 2) & 1)
            return c2_

        lax.fori_loop(0, n_vregs, vreg_body, 0)
        pltpu.sync_copy(o1_v, o1_hbm.at[pl.ds(start, ch)])
        pltpu.sync_copy(o2_v, o2_hbm.at[pl.ds(start, ch)])
        pltpu.sync_copy(o3_v, o3_hbm.at[pl.ds(start, ch)])
        return carry

    lax.fori_loop(0, nchunk, chunk_body, 0)


def _copy_block(x_ref, o_ref):
    o_ref[...] = x_ref[...]


def _dma_copy_body(x_hbm, o_hbm, *sems):
    n = len(sems)
    rows = x_hbm.shape[0] // n
    cps = [
        pltpu.make_async_copy(x_hbm.at[pl.ds(k * rows, rows)],
                              o_hbm.at[pl.ds(k * rows, rows)], sems[k])
        for k in range(n)
    ]
    for cp in cps:
        cp.start()
    for cp in cps:
        cp.wait()


def kernel(parent_coords, child_indices, parent_features):
    m = parent_coords.shape[0]
    assert m % 16 == 0 and m // _NW >= 16
    rows_per_w = (m // _NW) // 16 * 16
    rows_last = m - (_NW - 1) * rows_per_w
    ch = min(8192, rows_per_w) // 16 * 16
    nchunk = -(-rows_last // ch)

    ci = child_indices.astype(jnp.int32)

    body = functools.partial(_fcg_body, rows_per_w, rows_last, ch, nchunk)
    o1, o2, o3 = pl.kernel(
        body,
        out_type=[jax.ShapeDtypeStruct((m,), jnp.int32) for _ in range(3)],
        mesh=plsc.VectorSubcoreMesh(core_axis_name="c", subcore_axis_name="s"),
        scratch_types=[pltpu.VMEM((ch,), jnp.int32) for _ in range(7)],
    )(ci, parent_coords[:, 1], parent_coords[:, 2], parent_coords[:, 3])
    c, mm = parent_features.shape[1], m
    bn = 57344
    pfT = pl.pallas_call(
        _copy_block,
        out_shape=jax.ShapeDtypeStruct((c, mm), parent_features.dtype),
        grid=(pl.cdiv(mm, bn),),
        in_specs=[pl.BlockSpec((c, bn), lambda i: (0, i))],
        out_specs=pl.BlockSpec((c, bn), lambda i: (0, i)),
    )(parent_features.T)
    pf = pfT.T
    o1, o2, o3, pf = lax.optimization_barrier((o1, o2, o3, pf))
    out = jnp.stack([parent_coords[:, 0], o1, o2, o3], axis=1)
    return out, pf
